# Initial kernel scaffold; baseline (speedup 1.0000x reference)
import jax
import jax.numpy as jnp
from jax.experimental import pallas as pl


def kernel(x, word_vectors, W_proj, Wt0, bt0, Wg0, bg0, Wt1, bt1, Wg1, bg1):
    raise NotImplementedError("write your pallas kernel here")


# speedup vs baseline: 3.5018x; 3.5018x over previous
import functools

import jax
import jax.numpy as jnp
from jax import lax
from jax.experimental import pallas as pl
from jax.experimental.pallas import tpu as pltpu
from jax.experimental.pallas import tpu_sc as plsc

D = 128
NTOK = 1024 * 200

NC, NS = 2, 16
NW = NC * NS
CHUNK = 80
ROWS_PER_W = NTOK // NW
CHUNKS_PER_W = ROWS_PER_W // CHUNK


def _gather_body(idx_hbm, table_hbm, out_hbm, idx_v, rows_v, sem):
    wid = lax.axis_index("s") * NC + lax.axis_index("c")
    row0 = wid * CHUNKS_PER_W
    pltpu.sync_copy(idx_hbm.at[pl.ds(row0, CHUNKS_PER_W)], idx_v)

    def chunk(j, _):
        pltpu.async_copy(table_hbm.at[idx_v.at[j]], rows_v, sem).wait()
        tok = (row0 + j) * CHUNK
        pltpu.sync_copy(rows_v, out_hbm.at[pl.ds(tok, CHUNK)])
        return _

    lax.fori_loop(0, CHUNKS_PER_W, chunk, None)


_sc_gather = functools.partial(
    pl.kernel,
    out_type=jax.ShapeDtypeStruct((NTOK, D), jnp.float32),
    mesh=plsc.VectorSubcoreMesh(core_axis_name="c", subcore_axis_name="s"),
    scratch_types=[
        pltpu.VMEM((CHUNKS_PER_W, CHUNK), jnp.int32),
        pltpu.VMEM((CHUNK, D), jnp.float32),
        pltpu.SemaphoreType.DMA,
    ],
)(_gather_body)


def _mm(a, b_ref):
    return lax.dot_general(a, b_ref[...], (((1,), (1,)), ((), ())),
                           preferred_element_type=jnp.float32)


def _highway_body(x_ref, wp, wt0, bt0, wg0, bg0, wt1, bt1, wg1, bg1, o_ref):
    h = _mm(x_ref[...], wp)
    for wt, bt, wg, bg in ((wt0, bt0, wg0, bg0), (wt1, bt1, wg1, bg1)):
        g = jax.nn.sigmoid(_mm(h, wg) + bg[...])
        t = jnp.maximum(_mm(h, wt) + bt[...], 0.0)
        h = g * t + (1.0 - g) * h
    o_ref[...] = h


TILE = 1024


def _highway(emb, wp, wt0, bt0, wg0, bg0, wt1, bt1, wg1, bg1):
    n = emb.shape[0]
    grid = n // TILE
    wspec = pl.BlockSpec((D, D), lambda i: (0, 0))
    bspec = pl.BlockSpec((1, D), lambda i: (0, 0))
    return pl.pallas_call(
        _highway_body,
        grid=(grid,),
        in_specs=[pl.BlockSpec((TILE, D), lambda i: (i, 0)),
                  wspec, wspec, bspec, wspec, bspec,
                  wspec, bspec, wspec, bspec],
        out_specs=pl.BlockSpec((TILE, D), lambda i: (i, 0)),
        out_shape=jax.ShapeDtypeStruct((n, D), jnp.float32),
        compiler_params=pltpu.CompilerParams(
            dimension_semantics=("arbitrary",)),
    )(emb, wp, wt0, bt0, wg0, bg0, wt1, bt1, wg1, bg1)


def kernel(x, word_vectors, W_proj, Wt0, bt0, Wg0, bg0, Wt1, bt1, Wg1, bg1):
    B, L = x.shape
    idx = x.reshape(NW * CHUNKS_PER_W, CHUNK).astype(jnp.int32)
    emb = _sc_gather(idx, word_vectors)
    out = _highway(emb, W_proj,
                   Wt0, bt0.reshape(1, D), Wg0, bg0.reshape(1, D),
                   Wt1, bt1.reshape(1, D), Wg1, bg1.reshape(1, D))
    return out.reshape(B, L, D)


# speedup vs baseline: 4.4138x; 1.2604x over previous
import functools

import jax
import jax.numpy as jnp
from jax import lax
from jax.experimental import pallas as pl
from jax.experimental.pallas import tpu as pltpu
from jax.experimental.pallas import tpu_sc as plsc

D = 128
NTOK = 1024 * 200

NC, NS = 2, 16
NW = NC * NS

SEG = 5
NTOK_SEG = NTOK // SEG
CHUNK = 80
IDX_ROWS = NTOK // CHUNK
IDX_ROWS_SEG = IDX_ROWS // SEG
CPW = IDX_ROWS_SEG // NW


def _gather_body(idx_hbm, table_hbm, out_hbm, idx_v, rows_v, sem):
    wid = lax.axis_index("s") * NC + lax.axis_index("c")
    row0 = wid * CPW
    pltpu.sync_copy(idx_hbm.at[pl.ds(row0, CPW)], idx_v)
    for j in range(CPW):
        pltpu.async_copy(table_hbm.at[idx_v.at[j]], rows_v, sem).wait()
        pltpu.sync_copy(rows_v, out_hbm.at[pl.ds((row0 + j) * CHUNK, CHUNK)])


_sc_gather = functools.partial(
    pl.kernel,
    out_type=jax.ShapeDtypeStruct((NTOK_SEG, D), jnp.float32),
    mesh=plsc.VectorSubcoreMesh(core_axis_name="c", subcore_axis_name="s"),
    scratch_types=[
        pltpu.VMEM((CPW, CHUNK), jnp.int32),
        pltpu.VMEM((CHUNK, D), jnp.float32),
        pltpu.SemaphoreType.DMA,
    ],
)(_gather_body)


def _mm(a, b):
    return lax.dot_general(a, b, (((1,), (1,)), ((), ())),
                           preferred_element_type=jnp.float32)


TILE = 1024
TILES_SEG = NTOK_SEG // TILE


def _hw_body(carry, x_ref, wp, w0, b0, w1, b1, o_ref):
    del carry
    h = _mm(x_ref[...], wp[...])
    for w, b in ((w0, b0), (w1, b1)):
        tg = _mm(h, w[...]) + b[...]
        t = jnp.maximum(tg[:, :D], 0.0)
        g = jax.nn.sigmoid(tg[:, D:])
        h = g * t + (1.0 - g) * h
    o_ref[...] = h


def _hw_seg(k, carry, emb, wp, w0, b0, w1, b1):
    tile_spec = pl.BlockSpec((TILE, D), lambda i: (i, 0))
    wspec = pl.BlockSpec(None, lambda i: (0, 0))
    body = _hw_body
    in_specs = [pl.BlockSpec(memory_space=pl.ANY),
                tile_spec, wspec, wspec, wspec, wspec, wspec]
    args = (carry, emb, wp, w0, b0, w1, b1)
    if carry is None:
        body = functools.partial(_hw_body, None)
        in_specs = in_specs[1:]
        args = args[1:]
    return pl.pallas_call(
        body,
        grid=(TILES_SEG,),
        in_specs=in_specs,
        out_specs=pl.BlockSpec((TILE, D), lambda i: (i + TILES_SEG * k, 0)),
        out_shape=jax.ShapeDtypeStruct((NTOK, D), jnp.float32),
        input_output_aliases={} if carry is None else {0: 0},
        compiler_params=pltpu.CompilerParams(
            dimension_semantics=("arbitrary",)),
    )(*args)


def kernel(x, word_vectors, W_proj, Wt0, bt0, Wg0, bg0, Wt1, bt1, Wg1, bg1):
    B, L = x.shape
    idx = x.reshape(IDX_ROWS, CHUNK).astype(jnp.int32)
    w0 = jnp.concatenate([Wt0, Wg0], axis=0)
    w1 = jnp.concatenate([Wt1, Wg1], axis=0)
    b0 = jnp.concatenate([bt0, bg0]).reshape(1, 2 * D)
    b1 = jnp.concatenate([bt1, bg1]).reshape(1, 2 * D)

    embs = [_sc_gather(idx[k * IDX_ROWS_SEG:(k + 1) * IDX_ROWS_SEG],
                       word_vectors)
            for k in range(SEG)]
    out = None
    for k in range(SEG):
        out = _hw_seg(k, out, embs[k], W_proj, w0, b0, w1, b1)
    return out.reshape(B, L, D)


# speedup vs baseline: 4.4451x; 1.0071x over previous
import functools

import jax
import jax.numpy as jnp
from jax import lax
from jax.experimental import pallas as pl
from jax.experimental.pallas import tpu as pltpu
from jax.experimental.pallas import tpu_sc as plsc

D = 128
NTOK = 1024 * 200

NC, NS = 2, 16
NW = NC * NS

SEG = 5
NTOK_SEG = NTOK // SEG
CHUNK = 80
IDX_ROWS = NTOK // CHUNK
IDX_ROWS_SEG = IDX_ROWS // SEG
CPW = IDX_ROWS_SEG // NW


NBUF = 3


def _gather_body(idx_hbm, table_hbm, out_hbm, idx_v,
                 rows0, rows1, rows2, gs0, gs1, gs2, ws0, ws1, ws2):
    wid = lax.axis_index("s") * NC + lax.axis_index("c")
    row0 = wid * CPW
    bufs = (rows0, rows1, rows2)
    gsems = (gs0, gs1, gs2)
    wsems = (ws0, ws1, ws2)
    pltpu.sync_copy(idx_hbm.at[pl.ds(row0, CPW)], idx_v)

    def fire_gather(j):
        b = j % NBUF
        return pltpu.async_copy(table_hbm.at[idx_v.at[j]], bufs[b], gsems[b])

    gh = [None] * NBUF
    wh = [None] * NBUF
    for j in range(min(2, CPW)):
        gh[j % NBUF] = fire_gather(j)
    for j in range(CPW):
        b = j % NBUF
        gh[b].wait()
        wh[b] = pltpu.async_copy(
            bufs[b], out_hbm.at[pl.ds((row0 + j) * CHUNK, CHUNK)], wsems[b])
        nxt = j + 2
        if nxt < CPW:
            nb = nxt % NBUF
            if wh[nb] is not None:
                wh[nb].wait()
            gh[nb] = fire_gather(nxt)
    for b in range(NBUF):
        if wh[b] is not None:
            wh[b].wait()


_sc_gather = functools.partial(
    pl.kernel,
    out_type=jax.ShapeDtypeStruct((NTOK_SEG, D), jnp.float32),
    mesh=plsc.VectorSubcoreMesh(core_axis_name="c", subcore_axis_name="s"),
    scratch_types=[
        pltpu.VMEM((CPW, CHUNK), jnp.int32),
        pltpu.VMEM((CHUNK, D), jnp.float32),
        pltpu.VMEM((CHUNK, D), jnp.float32),
        pltpu.VMEM((CHUNK, D), jnp.float32),
        pltpu.SemaphoreType.DMA,
        pltpu.SemaphoreType.DMA,
        pltpu.SemaphoreType.DMA,
        pltpu.SemaphoreType.DMA,
        pltpu.SemaphoreType.DMA,
        pltpu.SemaphoreType.DMA,
    ],
)(_gather_body)


def _mm(a, b):
    return lax.dot_general(a, b, (((1,), (1,)), ((), ())),
                           preferred_element_type=jnp.float32)


TILE = 1024
TILES_SEG = NTOK_SEG // TILE


def _hw_body(carry, x_ref, wp, w0, b0, w1, b1, o_ref):
    del carry
    h = _mm(x_ref[...], wp[...])
    for w, b in ((w0, b0), (w1, b1)):
        tg = _mm(h, w[...]) + b[...]
        t = jnp.maximum(tg[:, :D], 0.0)
        g = jax.nn.sigmoid(tg[:, D:])
        h = g * t + (1.0 - g) * h
    o_ref[...] = h


def _hw_seg(k, carry, emb, wp, w0, b0, w1, b1):
    tile_spec = pl.BlockSpec((TILE, D), lambda i: (i, 0))
    wspec = pl.BlockSpec(None, lambda i: (0, 0))
    body = _hw_body
    in_specs = [pl.BlockSpec(memory_space=pl.ANY),
                tile_spec, wspec, wspec, wspec, wspec, wspec]
    args = (carry, emb, wp, w0, b0, w1, b1)
    if carry is None:
        body = functools.partial(_hw_body, None)
        in_specs = in_specs[1:]
        args = args[1:]
    return pl.pallas_call(
        body,
        grid=(TILES_SEG,),
        in_specs=in_specs,
        out_specs=pl.BlockSpec((TILE, D), lambda i: (i + TILES_SEG * k, 0)),
        out_shape=jax.ShapeDtypeStruct((NTOK, D), jnp.float32),
        input_output_aliases={} if carry is None else {0: 0},
        compiler_params=pltpu.CompilerParams(
            dimension_semantics=("arbitrary",)),
    )(*args)


def kernel(x, word_vectors, W_proj, Wt0, bt0, Wg0, bg0, Wt1, bt1, Wg1, bg1):
    B, L = x.shape
    idx = x.reshape(IDX_ROWS, CHUNK).astype(jnp.int32)
    w0 = jnp.concatenate([Wt0, Wg0], axis=0)
    w1 = jnp.concatenate([Wt1, Wg1], axis=0)
    b0 = jnp.concatenate([bt0, bg0]).reshape(1, 2 * D)
    b1 = jnp.concatenate([bt1, bg1]).reshape(1, 2 * D)

    embs = [_sc_gather(idx[k * IDX_ROWS_SEG:(k + 1) * IDX_ROWS_SEG],
                       word_vectors)
            for k in range(SEG)]
    out = None
    for k in range(SEG):
        out = _hw_seg(k, out, embs[k], W_proj, w0, b0, w1, b1)
    return out.reshape(B, L, D)


# speedup vs baseline: 5.6903x; 1.2801x over previous
import functools

import jax
import jax.numpy as jnp
from jax import lax
from jax.experimental import pallas as pl
from jax.experimental.pallas import tpu as pltpu
from jax.experimental.pallas import tpu_sc as plsc

D = 128
NTOK = 1024 * 200

NC, NS = 2, 16
NW = NC * NS

SEG = 5
NTOK_SEG = NTOK // SEG
CHUNK = 80
IDX_ROWS = NTOK // CHUNK
IDX_ROWS_SEG = IDX_ROWS // SEG
CPW = IDX_ROWS_SEG // NW


NBUF = 3


def _gather_body(idx_hbm, table_hbm, out_hbm, idx_v,
                 rows0, rows1, rows2, gs0, gs1, gs2, ws0, ws1, ws2):
    wid = lax.axis_index("s") * NC + lax.axis_index("c")
    row0 = wid * CPW
    bufs = (rows0, rows1, rows2)
    gsems = (gs0, gs1, gs2)
    wsems = (ws0, ws1, ws2)
    pltpu.sync_copy(idx_hbm.at[pl.ds(row0, CPW)], idx_v)

    def fire_gather(j):
        b = j % NBUF
        return pltpu.async_copy(table_hbm.at[idx_v.at[j]], bufs[b], gsems[b])

    gh = [None] * NBUF
    wh = [None] * NBUF
    for j in range(min(2, CPW)):
        gh[j % NBUF] = fire_gather(j)
    for j in range(CPW):
        b = j % NBUF
        gh[b].wait()
        wh[b] = pltpu.async_copy(
            bufs[b], out_hbm.at[pl.ds((row0 + j) * CHUNK, CHUNK)], wsems[b])
        nxt = j + 2
        if nxt < CPW:
            nb = nxt % NBUF
            if wh[nb] is not None:
                wh[nb].wait()
            gh[nb] = fire_gather(nxt)
    for b in range(NBUF):
        if wh[b] is not None:
            wh[b].wait()


_sc_gather = functools.partial(
    pl.kernel,
    out_type=jax.ShapeDtypeStruct((NTOK_SEG, D), jnp.float32),
    mesh=plsc.VectorSubcoreMesh(core_axis_name="c", subcore_axis_name="s"),
    scratch_types=[
        pltpu.VMEM((CPW, CHUNK), jnp.int32),
        pltpu.VMEM((CHUNK, D), jnp.float32),
        pltpu.VMEM((CHUNK, D), jnp.float32),
        pltpu.VMEM((CHUNK, D), jnp.float32),
        pltpu.SemaphoreType.DMA,
        pltpu.SemaphoreType.DMA,
        pltpu.SemaphoreType.DMA,
        pltpu.SemaphoreType.DMA,
        pltpu.SemaphoreType.DMA,
        pltpu.SemaphoreType.DMA,
    ],
)(_gather_body)


def _mm(a, b):
    return lax.dot_general(a.astype(jnp.bfloat16), b.astype(jnp.bfloat16),
                           (((1,), (1,)), ((), ())),
                           preferred_element_type=jnp.float32)


TILE = 2048
TILES_SEG = NTOK_SEG // TILE


def _hw_body(carry, x_ref, wp, w0, b0, w1, b1, o_ref):
    del carry
    h = _mm(x_ref[...], wp[...])
    for w, b in ((w0, b0), (w1, b1)):
        tg = _mm(h, w[...]) + b[...]
        t = jnp.maximum(tg[:, :D], 0.0)
        g = jax.nn.sigmoid(tg[:, D:])
        h = g * t + (1.0 - g) * h
    o_ref[...] = h


def _hw_seg(k, carry, emb, wp, w0, b0, w1, b1):
    tile_spec = pl.BlockSpec((TILE, D), lambda i: (i, 0))
    wspec = pl.BlockSpec(None, lambda i: (0, 0))
    body = _hw_body
    in_specs = [pl.BlockSpec(memory_space=pl.ANY),
                tile_spec, wspec, wspec, wspec, wspec, wspec]
    args = (carry, emb, wp, w0, b0, w1, b1)
    if carry is None:
        body = functools.partial(_hw_body, None)
        in_specs = in_specs[1:]
        args = args[1:]
    return pl.pallas_call(
        body,
        grid=(TILES_SEG,),
        in_specs=in_specs,
        out_specs=pl.BlockSpec((TILE, D), lambda i: (i + TILES_SEG * k, 0)),
        out_shape=jax.ShapeDtypeStruct((NTOK, D), jnp.float32),
        input_output_aliases={} if carry is None else {0: 0},
        compiler_params=pltpu.CompilerParams(
            dimension_semantics=("arbitrary",)),
    )(*args)


def kernel(x, word_vectors, W_proj, Wt0, bt0, Wg0, bg0, Wt1, bt1, Wg1, bg1):
    B, L = x.shape
    idx = x.reshape(IDX_ROWS, CHUNK).astype(jnp.int32)
    w0 = jnp.concatenate([Wt0, Wg0], axis=0).astype(jnp.bfloat16)
    w1 = jnp.concatenate([Wt1, Wg1], axis=0).astype(jnp.bfloat16)
    wp = W_proj.astype(jnp.bfloat16)
    b0 = jnp.concatenate([bt0, bg0]).reshape(1, 2 * D)
    b1 = jnp.concatenate([bt1, bg1]).reshape(1, 2 * D)

    embs = [_sc_gather(idx[k * IDX_ROWS_SEG:(k + 1) * IDX_ROWS_SEG],
                       word_vectors)
            for k in range(SEG)]
    out = None
    for k in range(SEG):
        out = _hw_seg(k, out, embs[k], wp, w0, b0, w1, b1)
    return out.reshape(B, L, D)


# speedup vs baseline: 6.0978x; 1.0716x over previous
import functools

import jax
import jax.numpy as jnp
import numpy as np
from jax import lax
from jax.experimental import pallas as pl
from jax.experimental.pallas import tpu as pltpu
from jax.experimental.pallas import tpu_sc as plsc

D = 128
NTOK = 1024 * 200

NC, NS = 2, 16
NW = NC * NS

SEG = 5
NTOK_SEG = NTOK // SEG
CHUNK = 80
IDX_ROWS = NTOK // CHUNK
IDX_ROWS_SEG = IDX_ROWS // SEG
CPW = IDX_ROWS_SEG // NW


NBUF = 3


def _bf16_pack(src, dst):
    def row(r, carry):
        for c in range(0, D, 32):
            dst[r, pl.ds(c, 32)] = plsc.pack(
                src[r, pl.ds(c, 16)], src[r, pl.ds(c + 16, 16)],
                format=plsc.PackFormat.INTERLEAVED)
        return carry
    lax.fori_loop(0, CHUNK, row, None)


def _gather_body(idx_hbm, table_hbm, out_hbm, idx_v,
                 rows0, rows1, rows2, bf0, bf1, bf2,
                 gs0, gs1, gs2, ws0, ws1, ws2):
    wid = lax.axis_index("s") * NC + lax.axis_index("c")
    row0 = wid * CPW
    bufs = (rows0, rows1, rows2)
    bfbufs = (bf0, bf1, bf2)
    gsems = (gs0, gs1, gs2)
    wsems = (ws0, ws1, ws2)
    pltpu.sync_copy(idx_hbm.at[pl.ds(row0, CPW)], idx_v)

    def fire_gather(j):
        b = j % NBUF
        return pltpu.async_copy(table_hbm.at[idx_v.at[j]], bufs[b], gsems[b])

    gh = [None] * NBUF
    wh = [None] * NBUF
    for j in range(min(2, CPW)):
        gh[j % NBUF] = fire_gather(j)
    for j in range(CPW):
        b = j % NBUF
        gh[b].wait()
        if wh[b] is not None:
            wh[b].wait()
        _bf16_pack(bufs[b], bfbufs[b])
        wh[b] = pltpu.async_copy(
            bfbufs[b], out_hbm.at[pl.ds((row0 + j) * CHUNK, CHUNK)], wsems[b])
        nxt = j + 2
        if nxt < CPW:
            gh[nxt % NBUF] = fire_gather(nxt)
    for b in range(NBUF):
        if wh[b] is not None:
            wh[b].wait()


_sc_gather = functools.partial(
    pl.kernel,
    out_type=jax.ShapeDtypeStruct((NTOK_SEG, D), jnp.bfloat16),
    mesh=plsc.VectorSubcoreMesh(core_axis_name="c", subcore_axis_name="s"),
    compiler_params=pltpu.CompilerParams(needs_layout_passes=False),
    scratch_types=[
        pltpu.VMEM((CPW, CHUNK), jnp.int32),
        pltpu.VMEM((CHUNK, D), jnp.float32),
        pltpu.VMEM((CHUNK, D), jnp.float32),
        pltpu.VMEM((CHUNK, D), jnp.float32),
        pltpu.VMEM((CHUNK, D), jnp.bfloat16),
        pltpu.VMEM((CHUNK, D), jnp.bfloat16),
        pltpu.VMEM((CHUNK, D), jnp.bfloat16),
        pltpu.SemaphoreType.DMA,
        pltpu.SemaphoreType.DMA,
        pltpu.SemaphoreType.DMA,
        pltpu.SemaphoreType.DMA,
        pltpu.SemaphoreType.DMA,
        pltpu.SemaphoreType.DMA,
    ],
)(_gather_body)


def _mm(a, b):
    return lax.dot_general(a.astype(jnp.bfloat16), b.astype(jnp.bfloat16),
                           (((1,), (1,)), ((), ())),
                           preferred_element_type=jnp.float32)


TILE = 2048
TILES_SEG = NTOK_SEG // TILE


def _hw_body(carry, x_ref, wp, w0, b0, w1, b1, o_ref):
    del carry
    h = _mm(x_ref[...], wp[...])
    for w, b in ((w0, b0), (w1, b1)):
        tg = _mm(h, w[...]) + b[...]
        t = jnp.maximum(tg[:, :D], 0.0)
        g = jax.nn.sigmoid(tg[:, D:])
        h = g * t + (1.0 - g) * h
    o_ref[...] = h


def _hw_seg(k, carry, emb, wp, w0, b0, w1, b1):
    tile_spec = pl.BlockSpec((TILE, D), lambda i: (i, 0))
    wspec = pl.BlockSpec(None, lambda i: (0, 0))
    body = _hw_body
    in_specs = [pl.BlockSpec(memory_space=pl.ANY),
                tile_spec, wspec, wspec, wspec, wspec, wspec]
    args = (carry, emb, wp, w0, b0, w1, b1)
    if carry is None:
        body = functools.partial(_hw_body, None)
        in_specs = in_specs[1:]
        args = args[1:]
    return pl.pallas_call(
        body,
        grid=(TILES_SEG,),
        in_specs=in_specs,
        out_specs=pl.BlockSpec((TILE, D), lambda i: (i + TILES_SEG * k, 0)),
        out_shape=jax.ShapeDtypeStruct((NTOK, D), jnp.float32),
        input_output_aliases={} if carry is None else {0: 0},
        compiler_params=pltpu.CompilerParams(
            dimension_semantics=("arbitrary",)),
    )(*args)


def kernel(x, word_vectors, W_proj, Wt0, bt0, Wg0, bg0, Wt1, bt1, Wg1, bg1):
    B, L = x.shape
    idx = x.reshape(IDX_ROWS, CHUNK).astype(jnp.int32)
    w0 = jnp.concatenate([Wt0, Wg0], axis=0).astype(jnp.bfloat16)
    w1 = jnp.concatenate([Wt1, Wg1], axis=0).astype(jnp.bfloat16)
    grp = np.arange(32).reshape(2, 16).T.ravel()
    perm = (np.arange(0, D, 32)[:, None] + grp[None, :]).ravel()
    wp = W_proj[:, perm].astype(jnp.bfloat16)
    b0 = jnp.concatenate([bt0, bg0]).reshape(1, 2 * D)
    b1 = jnp.concatenate([bt1, bg1]).reshape(1, 2 * D)

    embs = [_sc_gather(idx[k * IDX_ROWS_SEG:(k + 1) * IDX_ROWS_SEG],
                       word_vectors)
            for k in range(SEG)]
    out = None
    for k in range(SEG):
        out = _hw_seg(k, out, embs[k], wp, w0, b0, w1, b1)
    return out.reshape(B, L, D)


# speedup vs baseline: 6.1083x; 1.0017x over previous
import functools

import jax
import jax.numpy as jnp
import numpy as np
from jax import lax
from jax.experimental import pallas as pl
from jax.experimental.pallas import tpu as pltpu
from jax.experimental.pallas import tpu_sc as plsc

D = 128
NTOK = 1024 * 200

NC, NS = 2, 16
NW = NC * NS

SEG = 5
NTOK_SEG = NTOK // SEG
CHUNK = 80
IDX_ROWS = NTOK // CHUNK
IDX_ROWS_SEG = IDX_ROWS // SEG
CPW = IDX_ROWS_SEG // NW


NBUF = 6


PRIME = 3


def _bf16_pack(src, dst):
    def row(r, carry):
        for c in range(0, D, 32):
            dst[r, pl.ds(c, 32)] = plsc.pack(
                src[r, pl.ds(c, 16)], src[r, pl.ds(c + 16, 16)],
                format=plsc.PackFormat.INTERLEAVED)
        return carry
    lax.fori_loop(0, CHUNK, row, None)


def _gather_body(idx_hbm, table_hbm, out_hbm, idx_v, *scratch):
    wid = lax.axis_index("s") * NC + lax.axis_index("c")
    row0 = wid * CPW
    bufs = scratch[0:NBUF]
    bfbufs = scratch[NBUF:2 * NBUF]
    gsems = scratch[2 * NBUF:3 * NBUF]
    wsems = scratch[3 * NBUF:4 * NBUF]
    pltpu.sync_copy(idx_hbm.at[pl.ds(row0, CPW)], idx_v)

    def fire_gather(j):
        b = j % NBUF
        return pltpu.async_copy(table_hbm.at[idx_v.at[j]], bufs[b], gsems[b])

    gh = [None] * NBUF
    wh = [None] * NBUF
    for j in range(min(PRIME, CPW)):
        gh[j % NBUF] = fire_gather(j)
    for j in range(CPW):
        b = j % NBUF
        gh[b].wait()
        nxt = j + PRIME
        if nxt < CPW:
            nb = nxt % NBUF
            if wh[nb] is not None:
                wh[nb].wait()
                wh[nb] = None
            gh[nb] = fire_gather(nxt)
        if wh[b] is not None:
            wh[b].wait()
        _bf16_pack(bufs[b], bfbufs[b])
        wh[b] = pltpu.async_copy(
            bfbufs[b], out_hbm.at[pl.ds((row0 + j) * CHUNK, CHUNK)], wsems[b])
    for b in range(NBUF):
        if wh[b] is not None:
            wh[b].wait()


_sc_gather = functools.partial(
    pl.kernel,
    out_type=jax.ShapeDtypeStruct((NTOK_SEG, D), jnp.bfloat16),
    mesh=plsc.VectorSubcoreMesh(core_axis_name="c", subcore_axis_name="s"),
    compiler_params=pltpu.CompilerParams(needs_layout_passes=False),
    scratch_types=(
        [pltpu.VMEM((CPW, CHUNK), jnp.int32)]
        + [pltpu.VMEM((CHUNK, D), jnp.float32)] * NBUF
        + [pltpu.VMEM((CHUNK, D), jnp.bfloat16)] * NBUF
        + [pltpu.SemaphoreType.DMA] * (2 * NBUF)
    ),
)(_gather_body)


def _mm(a, b):
    return lax.dot_general(a.astype(jnp.bfloat16), b.astype(jnp.bfloat16),
                           (((1,), (1,)), ((), ())),
                           preferred_element_type=jnp.float32)


TILE = 2048
TILES_SEG = NTOK_SEG // TILE


def _hw_body(carry, x_ref, wp, w0, b0, w1, b1, o_ref):
    del carry
    h = _mm(x_ref[...], wp[...])
    for w, b in ((w0, b0), (w1, b1)):
        tg = _mm(h, w[...]) + b[...]
        t = jnp.maximum(tg[:, :D], 0.0)
        g = jax.nn.sigmoid(tg[:, D:])
        h = g * t + (1.0 - g) * h
    o_ref[...] = h


def _hw_seg(k, carry, emb, wp, w0, b0, w1, b1):
    tile_spec = pl.BlockSpec((TILE, D), lambda i: (i, 0))
    wspec = pl.BlockSpec(None, lambda i: (0, 0))
    body = _hw_body
    in_specs = [pl.BlockSpec(memory_space=pl.ANY),
                tile_spec, wspec, wspec, wspec, wspec, wspec]
    args = (carry, emb, wp, w0, b0, w1, b1)
    if carry is None:
        body = functools.partial(_hw_body, None)
        in_specs = in_specs[1:]
        args = args[1:]
    return pl.pallas_call(
        body,
        grid=(TILES_SEG,),
        in_specs=in_specs,
        out_specs=pl.BlockSpec((TILE, D), lambda i: (i + TILES_SEG * k, 0)),
        out_shape=jax.ShapeDtypeStruct((NTOK, D), jnp.float32),
        input_output_aliases={} if carry is None else {0: 0},
        compiler_params=pltpu.CompilerParams(
            dimension_semantics=("arbitrary",)),
    )(*args)


def kernel(x, word_vectors, W_proj, Wt0, bt0, Wg0, bg0, Wt1, bt1, Wg1, bg1):
    B, L = x.shape
    idx = x.reshape(IDX_ROWS, CHUNK).astype(jnp.int32)
    w0 = jnp.concatenate([Wt0, Wg0], axis=0).astype(jnp.bfloat16)
    w1 = jnp.concatenate([Wt1, Wg1], axis=0).astype(jnp.bfloat16)
    grp = np.arange(32).reshape(2, 16).T.ravel()
    perm = (np.arange(0, D, 32)[:, None] + grp[None, :]).ravel()
    wp = W_proj[:, perm].astype(jnp.bfloat16)
    b0 = jnp.concatenate([bt0, bg0]).reshape(1, 2 * D)
    b1 = jnp.concatenate([bt1, bg1]).reshape(1, 2 * D)

    embs = [_sc_gather(idx[k * IDX_ROWS_SEG:(k + 1) * IDX_ROWS_SEG],
                       word_vectors)
            for k in range(SEG)]
    out = None
    for k in range(SEG):
        out = _hw_seg(k, out, embs[k], wp, w0, b0, w1, b1)
    return out.reshape(B, L, D)


# speedup vs baseline: 6.4271x; 1.0522x over previous
import functools

import jax
import jax.numpy as jnp
import numpy as np
from jax import lax
from jax.experimental import pallas as pl
from jax.experimental.pallas import tpu as pltpu
from jax.experimental.pallas import tpu_sc as plsc

D = 128
NTOK = 1024 * 200

NC, NS = 2, 16
NW = NC * NS

SEG = 5
NTOK_SEG = NTOK // SEG
CHUNK = 80
IDX_ROWS = NTOK // CHUNK
IDX_ROWS_SEG = IDX_ROWS // SEG
CPW = IDX_ROWS_SEG // NW


NBUF = 6


PRIME = 3


def _bf16_pack(src, dst):
    @plsc.parallel_loop(0, CHUNK, 1, unroll=1)
    def _(r):
        for c in range(0, D, 32):
            dst[r, pl.ds(c, 32)] = plsc.pack(
                src[r, pl.ds(c, 16)], src[r, pl.ds(c + 16, 16)],
                format=plsc.PackFormat.INTERLEAVED)


def _gather_body(idx_hbm, table_hbm, out_hbm, idx_v, *scratch):
    wid = lax.axis_index("s") * NC + lax.axis_index("c")
    row0 = wid * CPW
    bufs = scratch[0:NBUF]
    bfbufs = scratch[NBUF:2 * NBUF]
    gsems = scratch[2 * NBUF:3 * NBUF]
    wsems = scratch[3 * NBUF:4 * NBUF]
    pltpu.sync_copy(idx_hbm.at[pl.ds(row0, CPW)], idx_v)

    def fire_gather(j):
        b = j % NBUF
        return pltpu.async_copy(table_hbm.at[idx_v.at[j]], bufs[b], gsems[b])

    gh = [None] * NBUF
    wh = [None] * NBUF
    for j in range(min(PRIME, CPW)):
        gh[j % NBUF] = fire_gather(j)
    for j in range(CPW):
        b = j % NBUF
        gh[b].wait()
        nxt = j + PRIME
        if nxt < CPW:
            nb = nxt % NBUF
            if wh[nb] is not None:
                wh[nb].wait()
                wh[nb] = None
            gh[nb] = fire_gather(nxt)
        if wh[b] is not None:
            wh[b].wait()
        _bf16_pack(bufs[b], bfbufs[b])
        wh[b] = pltpu.async_copy(
            bfbufs[b], out_hbm.at[pl.ds((row0 + j) * CHUNK, CHUNK)], wsems[b])
    for b in range(NBUF):
        if wh[b] is not None:
            wh[b].wait()


_sc_gather = functools.partial(
    pl.kernel,
    out_type=jax.ShapeDtypeStruct((NTOK_SEG, D), jnp.bfloat16),
    mesh=plsc.VectorSubcoreMesh(core_axis_name="c", subcore_axis_name="s"),
    compiler_params=pltpu.CompilerParams(needs_layout_passes=False),
    scratch_types=(
        [pltpu.VMEM((CPW, CHUNK), jnp.int32)]
        + [pltpu.VMEM((CHUNK, D), jnp.float32)] * NBUF
        + [pltpu.VMEM((CHUNK, D), jnp.bfloat16)] * NBUF
        + [pltpu.SemaphoreType.DMA] * (2 * NBUF)
    ),
)(_gather_body)


def _mm(a, b):
    return lax.dot_general(a.astype(jnp.bfloat16), b.astype(jnp.bfloat16),
                           (((1,), (1,)), ((), ())),
                           preferred_element_type=jnp.float32)


TILE = 2048
TILES_SEG = NTOK_SEG // TILE


def _hw_body(carry, x_ref, wp, w0, b0, w1, b1, o_ref):
    del carry
    h = _mm(x_ref[...], wp[...])
    for w, b in ((w0, b0), (w1, b1)):
        tg = _mm(h, w[...]) + b[...]
        t = jnp.maximum(tg[:, :D], 0.0)
        g = jax.nn.sigmoid(tg[:, D:])
        h = g * t + (1.0 - g) * h
    o_ref[...] = h


def _hw_seg(k, carry, emb, wp, w0, b0, w1, b1):
    tile_spec = pl.BlockSpec((TILE, D), lambda i: (i, 0))
    wspec = pl.BlockSpec(None, lambda i: (0, 0))
    body = _hw_body
    in_specs = [pl.BlockSpec(memory_space=pl.ANY),
                tile_spec, wspec, wspec, wspec, wspec, wspec]
    args = (carry, emb, wp, w0, b0, w1, b1)
    if carry is None:
        body = functools.partial(_hw_body, None)
        in_specs = in_specs[1:]
        args = args[1:]
    return pl.pallas_call(
        body,
        grid=(TILES_SEG,),
        in_specs=in_specs,
        out_specs=pl.BlockSpec((TILE, D), lambda i: (i + TILES_SEG * k, 0)),
        out_shape=jax.ShapeDtypeStruct((NTOK, D), jnp.float32),
        input_output_aliases={} if carry is None else {0: 0},
        compiler_params=pltpu.CompilerParams(
            dimension_semantics=("arbitrary",)),
    )(*args)


def kernel(x, word_vectors, W_proj, Wt0, bt0, Wg0, bg0, Wt1, bt1, Wg1, bg1):
    B, L = x.shape
    idx = x.reshape(IDX_ROWS, CHUNK).astype(jnp.int32)
    w0 = jnp.concatenate([Wt0, Wg0], axis=0).astype(jnp.bfloat16)
    w1 = jnp.concatenate([Wt1, Wg1], axis=0).astype(jnp.bfloat16)
    grp = np.arange(32).reshape(2, 16).T.ravel()
    perm = (np.arange(0, D, 32)[:, None] + grp[None, :]).ravel()
    wp = W_proj[:, perm].astype(jnp.bfloat16)
    b0 = jnp.concatenate([bt0, bg0]).reshape(1, 2 * D)
    b1 = jnp.concatenate([bt1, bg1]).reshape(1, 2 * D)

    embs = [_sc_gather(idx[k * IDX_ROWS_SEG:(k + 1) * IDX_ROWS_SEG],
                       word_vectors)
            for k in range(SEG)]
    out = None
    for k in range(SEG):
        out = _hw_seg(k, out, embs[k], wp, w0, b0, w1, b1)
    return out.reshape(B, L, D)


# speedup vs baseline: 7.4863x; 1.1648x over previous
import functools

import jax
import jax.numpy as jnp
import numpy as np
from jax import lax
from jax.experimental import pallas as pl
from jax.experimental.pallas import tpu as pltpu
from jax.experimental.pallas import tpu_sc as plsc

D = 128
NTOK = 1024 * 200

NC, NS = 2, 16
NW = NC * NS

SEG = 5
NTOK_SEG = NTOK // SEG
CHUNK = 80
IDX_ROWS = NTOK // CHUNK
IDX_ROWS_SEG = IDX_ROWS // SEG
CPW = IDX_ROWS_SEG // NW


NBUF = 6


PRIME = 3


def _bf16_pack(src, dst):
    @plsc.parallel_loop(0, CHUNK, 1, unroll=1)
    def _(r):
        for c in range(0, D, 32):
            dst[r, pl.ds(c, 32)] = plsc.pack(
                src[r, pl.ds(c, 16)], src[r, pl.ds(c + 16, 16)],
                format=plsc.PackFormat.INTERLEAVED)


def _gather_body(idx_hbm, table_hbm, out_hbm, idx_v, *scratch):
    wid = lax.axis_index("s") * NC + lax.axis_index("c")
    row0 = wid * CPW
    bufs = scratch[0:NBUF]
    bfbufs = scratch[NBUF:2 * NBUF]
    gsems = scratch[2 * NBUF:3 * NBUF]
    wsems = scratch[3 * NBUF:4 * NBUF]
    pltpu.sync_copy(idx_hbm.at[pl.ds(row0, CPW)], idx_v)

    def fire_gather(j):
        b = j % NBUF
        return pltpu.async_copy(table_hbm.at[idx_v.at[j]], bufs[b], gsems[b])

    gh = [None] * NBUF
    wh = [None] * NBUF
    for j in range(min(PRIME, CPW)):
        gh[j % NBUF] = fire_gather(j)
    for j in range(CPW):
        b = j % NBUF
        gh[b].wait()
        nxt = j + PRIME
        if nxt < CPW:
            nb = nxt % NBUF
            if wh[nb] is not None:
                wh[nb].wait()
                wh[nb] = None
            gh[nb] = fire_gather(nxt)
        if wh[b] is not None:
            wh[b].wait()
        _bf16_pack(bufs[b], bfbufs[b])
        wh[b] = pltpu.async_copy(
            bfbufs[b], out_hbm.at[pl.ds((row0 + j) * CHUNK, CHUNK)], wsems[b])
    for b in range(NBUF):
        if wh[b] is not None:
            wh[b].wait()


_sc_gather = functools.partial(
    pl.kernel,
    out_type=jax.ShapeDtypeStruct((NTOK_SEG, D), jnp.bfloat16),
    mesh=plsc.VectorSubcoreMesh(core_axis_name="c", subcore_axis_name="s"),
    compiler_params=pltpu.CompilerParams(needs_layout_passes=False),
    scratch_types=(
        [pltpu.VMEM((CPW, CHUNK), jnp.int32)]
        + [pltpu.VMEM((CHUNK, D), jnp.float32)] * NBUF
        + [pltpu.VMEM((CHUNK, D), jnp.bfloat16)] * NBUF
        + [pltpu.SemaphoreType.DMA] * (2 * NBUF)
    ),
)(_gather_body)


def _mm(a, b):
    return lax.dot_general(a.astype(jnp.bfloat16), b.astype(jnp.bfloat16),
                           (((1,), (1,)), ((), ())),
                           preferred_element_type=jnp.float32)


TILE = 4096
TILES_SEG = NTOK_SEG // TILE


def _hw_body(carry, x_ref, wp, w0, b0, w1, b1, o_ref):
    del carry
    h = _mm(x_ref[...], wp[...])
    for w, b in ((w0, b0), (w1, b1)):
        tg = _mm(h, w[...]) + b[...]
        t = jnp.maximum(tg[:, :D], 0.0)
        g = 0.5 * jnp.tanh(tg[:, D:] * 0.5) + 0.5
        h = h + g * (t - h)
    o_ref[...] = h


def _hw_seg(k, carry, emb, wp, w0, b0, w1, b1):
    tile_spec = pl.BlockSpec((TILE, D), lambda i: (i, 0))
    wspec = pl.BlockSpec(None, lambda i: (0, 0))
    body = _hw_body
    in_specs = [pl.BlockSpec(memory_space=pl.ANY),
                tile_spec, wspec, wspec, wspec, wspec, wspec]
    args = (carry, emb, wp, w0, b0, w1, b1)
    if carry is None:
        body = functools.partial(_hw_body, None)
        in_specs = in_specs[1:]
        args = args[1:]
    return pl.pallas_call(
        body,
        grid=(TILES_SEG,),
        in_specs=in_specs,
        out_specs=pl.BlockSpec((TILE, D), lambda i: (i + TILES_SEG * k, 0)),
        out_shape=jax.ShapeDtypeStruct((NTOK, D), jnp.float32),
        input_output_aliases={} if carry is None else {0: 0},
        compiler_params=pltpu.CompilerParams(
            dimension_semantics=("arbitrary",)),
    )(*args)


def kernel(x, word_vectors, W_proj, Wt0, bt0, Wg0, bg0, Wt1, bt1, Wg1, bg1):
    B, L = x.shape
    idx = x.reshape(IDX_ROWS, CHUNK).astype(jnp.int32)
    w0 = jnp.concatenate([Wt0, Wg0], axis=0).astype(jnp.bfloat16)
    w1 = jnp.concatenate([Wt1, Wg1], axis=0).astype(jnp.bfloat16)
    grp = np.arange(32).reshape(2, 16).T.ravel()
    perm = (np.arange(0, D, 32)[:, None] + grp[None, :]).ravel()
    wp = W_proj[:, perm].astype(jnp.bfloat16)
    b0 = jnp.concatenate([bt0, bg0]).reshape(1, 2 * D)
    b1 = jnp.concatenate([bt1, bg1]).reshape(1, 2 * D)

    embs = [_sc_gather(idx[k * IDX_ROWS_SEG:(k + 1) * IDX_ROWS_SEG],
                       word_vectors)
            for k in range(SEG)]
    out = None
    for k in range(SEG):
        out = _hw_seg(k, out, embs[k], wp, w0, b0, w1, b1)
    return out.reshape(B, L, D)
